# baseline (speedup 1.0000x reference)
import functools

import jax
import jax.numpy as jnp
from jax import lax
from jax.experimental import pallas as pl
from jax.experimental.pallas import tpu as pltpu
from jax.experimental.pallas import tpu_sc as plsc

_N = 10000
_E = 320000
_D = 128

_NC = 2
_NS = 16
_NW = _NC * _NS

_K = 128
_CPW = 80
_C = _CPW * _NW
_C0PW = 160
_C1PW = 0
_C1_BASE = _C0PW * _NS
_C_ALLOC = 2688
_E_PAD = _C * _K
_RPT = 632
_N_ACC = _RPT * _NS


def _mesh():
    return plsc.VectorSubcoreMesh(core_axis_name="c", subcore_axis_name="s")


def _deg_body(dst_hbm, z_hbm, ones_hbm, out_hbm, dst_v, ones_v, dacc_sh):
    cid = lax.axis_index("c")
    sid = lax.axis_index("s")
    wid = sid * _NC + cid
    pltpu.sync_copy(z_hbm, dacc_sh.at[pl.ds(sid * _RPT, _RPT)])
    pltpu.sync_copy(ones_hbm, ones_v)
    pltpu.sync_copy(dst_hbm.at[pl.ds(wid * _CPW, _CPW)], dst_v)
    plsc.subcore_barrier()

    def body(j, carry):
        pltpu.sync_copy(ones_v, dacc_sh.at[dst_v.at[j]], add=True)
        return carry

    lax.fori_loop(0, _CPW, body, 0)
    plsc.subcore_barrier()
    pltpu.sync_copy(dacc_sh.at[pl.ds(sid * _RPT, _RPT)],
                    out_hbm.at[cid, pl.ds(sid * _RPT, _RPT)])


def _sc_deg(dst2d, zeros_rows, ones_rows):
    kern = functools.partial(
        pl.kernel,
        out_type=jax.ShapeDtypeStruct((_NC, _N_ACC, _D), jnp.float32),
        mesh=_mesh(),
        scratch_types=[
            pltpu.VMEM((_CPW, _K), jnp.int32),
            pltpu.VMEM((_K, _D), jnp.float32),
            pltpu.VMEM_SHARED((_N_ACC, _D), jnp.float32),
        ],
    )(_deg_body)
    return kern(dst2d, zeros_rows, ones_rows)


def _agg_body(h_hbm, src_hbm, dst_hbm, z_hbm, out_hbm,
              src_v, dst_v, rows_v, acc_sh, sem):
    cid = lax.axis_index("c")
    sid = lax.axis_index("s")
    pltpu.sync_copy(z_hbm, acc_sh.at[pl.ds(sid * _RPT, _RPT)])
    plsc.subcore_barrier()
    base0 = jnp.where(cid == 0, sid * _C0PW, _C1_BASE)
    cnt = jnp.where(cid == 0, _CPW, 0)

    def outer(p, carry):
        pb = pl.multiple_of(base0 + p * _CPW, 8)
        pltpu.sync_copy(src_hbm.at[pl.ds(pb, _CPW)], src_v)
        pltpu.sync_copy(dst_hbm.at[pl.ds(pb, _CPW)], dst_v)

        def body(j, c2):
            pltpu.async_copy(h_hbm.at[src_v.at[j]], rows_v, sem).wait()
            pltpu.sync_copy(rows_v, acc_sh.at[dst_v.at[j]], add=True)
            return c2

        lax.fori_loop(0, cnt, body, 0)
        return carry

    lax.fori_loop(0, _C0PW // _CPW, outer, 0)
    plsc.subcore_barrier()
    pltpu.sync_copy(acc_sh.at[pl.ds(sid * _RPT, _RPT)],
                    out_hbm.at[cid, pl.ds(sid * _RPT, _RPT)])


def _sc_agg(h, src2d, dst2d, zeros_rows):
    kern = functools.partial(
        pl.kernel,
        out_type=jax.ShapeDtypeStruct((_NC, _N_ACC, _D), jnp.float32),
        mesh=_mesh(),
        scratch_types=[
            pltpu.VMEM((_CPW, _K), jnp.int32),
            pltpu.VMEM((_CPW, _K), jnp.int32),
            pltpu.VMEM((_K, _D), jnp.float32),
            pltpu.VMEM_SHARED((_N_ACC, _D), jnp.float32),
            pltpu.SemaphoreType.DMA,
        ],
    )(_agg_body)
    return kern(h, src2d, dst2d, zeros_rows)


def _mm_body(x_ref, w_ref, h_ref):
    h_ref[...] = jnp.dot(x_ref[...], w_ref[...],
                         preferred_element_type=jnp.float32,
                         precision=lax.Precision.HIGHEST)


def _tc_mm(x, w):
    return pl.pallas_call(
        _mm_body,
        out_shape=jax.ShapeDtypeStruct((_N, _D), jnp.float32),
    )(x, w)


def _scale_body(deg_ref, h_ref, dinv_ref, hp_ref):
    deg = deg_ref[0, :, 0:1] + deg_ref[1, :, 0:1] + 1.0
    dinv = lax.rsqrt(deg)
    dinv_ref[...] = dinv
    hp_ref[...] = h_ref[...] * dinv[:_N]


def _tc_scale(deg, h):
    return pl.pallas_call(
        _scale_body,
        out_shape=(jax.ShapeDtypeStruct((_N_ACC, 1), jnp.float32),
                   jax.ShapeDtypeStruct((_N, _D), jnp.float32)),
    )(deg, h)


def _mid_body(agg_ref, hp_ref, dinv_ref, b_ref, gw_ref, gb_ref, ga_ref,
              wn_ref, out_ref, hpn_ref):
    agg = agg_ref[0, :_N, :] + agg_ref[1, :_N, :] + hp_ref[...]
    t = agg * dinv_ref[:_N] + b_ref[...]
    mean = jnp.mean(t, axis=0, keepdims=True)
    u = t - ga_ref[...] * mean
    var = jnp.mean(u * u, axis=0, keepdims=True)
    g = u * lax.rsqrt(var + 1e-5) * gw_ref[...] + gb_ref[...]
    out_ref[...] = g
    e = jnp.where(g > 0, g, jnp.exp(jnp.minimum(g, 0.0)) - 1.0)
    hn = jnp.dot(e, wn_ref[...], preferred_element_type=jnp.float32,
                 precision=lax.Precision.HIGHEST)
    hpn_ref[...] = hn * dinv_ref[:_N]


def _tc_mid(agg, hp, dinv, b, gw, gb, ga, wnext):
    return pl.pallas_call(
        _mid_body,
        out_shape=(jax.ShapeDtypeStruct((_N, _D), jnp.float32),
                   jax.ShapeDtypeStruct((_N, _D), jnp.float32)),
    )(agg, hp, dinv, b, gw, gb, ga, wnext)


def _final_body(agg_ref, hp_ref, dinv_ref, b_ref, out_ref):
    agg = agg_ref[0, :_N, :] + agg_ref[1, :_N, :] + hp_ref[...]
    out_ref[...] = agg * dinv_ref[:_N] + b_ref[...]


def _tc_final(agg, hp, dinv, b):
    return pl.pallas_call(
        _final_body,
        out_shape=jax.ShapeDtypeStruct((_N, _D), jnp.float32),
    )(agg, hp, dinv, b)


def kernel(x, edge_index, W1, b1, W2, b2, W3, b3,
           gn_w1, gn_b1, gn_a1, gn_w2, gn_b2, gn_a2):
    src = edge_index[0]
    dst = edge_index[1]
    pad2 = _C_ALLOC * _K - _E
    src2d = jnp.concatenate(
        [src, jnp.zeros((pad2,), jnp.int32)]).reshape(_C_ALLOC, _K)
    dst2d = jnp.concatenate(
        [dst, jnp.full((pad2,), _N, jnp.int32)]).reshape(_C_ALLOC, _K)
    zeros_rows = jnp.zeros((_RPT, _D), jnp.float32)
    ones_rows = jnp.ones((_K, _D), jnp.float32)

    deg = _sc_deg(dst2d, zeros_rows, ones_rows)
    h1 = _tc_mm(x, W1)
    dinv, h1p = _tc_scale(deg, h1)

    agg1 = _sc_agg(h1p, src2d, dst2d, zeros_rows)
    out1, h2p = _tc_mid(agg1, h1p, dinv, b1.reshape(1, _D),
                        gn_w1.reshape(1, _D), gn_b1.reshape(1, _D),
                        gn_a1.reshape(1, _D), W2)

    agg2 = _sc_agg(h2p, src2d, dst2d, zeros_rows)
    out2, h3p = _tc_mid(agg2, h2p, dinv, b2.reshape(1, _D),
                        gn_w2.reshape(1, _D), gn_b2.reshape(1, _D),
                        gn_a2.reshape(1, _D), W3)

    agg3 = _sc_agg(h3p, src2d, dst2d, zeros_rows)
    out3 = _tc_final(agg3, h3p, dinv, b3.reshape(1, _D))

    return jnp.concatenate([out1, out2, out3], axis=-1)


# speedup vs baseline: 1.4400x; 1.4400x over previous
import functools

import jax
import jax.numpy as jnp
from jax import lax
from jax.experimental import pallas as pl
from jax.experimental.pallas import tpu as pltpu
from jax.experimental.pallas import tpu_sc as plsc

_N = 10000
_E = 320000
_D = 128

_NC = 2
_NS = 16
_NW = _NC * _NS

_K = 128
_CPW = 80
_C = _CPW * _NW
_C0PW = 128
_C1PW = 32
_C1_BASE = _C0PW * _NS
_C_ALLOC = 2688
_E_PAD = _C * _K
_RPT = 632
_N_ACC = _RPT * _NS


def _mesh():
    return plsc.VectorSubcoreMesh(core_axis_name="c", subcore_axis_name="s")


def _deg_body(dst_hbm, z_hbm, ones_hbm, out_hbm, dst_v, ones_v, dacc_sh):
    cid = lax.axis_index("c")
    sid = lax.axis_index("s")
    wid = sid * _NC + cid
    pltpu.sync_copy(z_hbm, dacc_sh.at[pl.ds(sid * _RPT, _RPT)])
    pltpu.sync_copy(ones_hbm, ones_v)
    pltpu.sync_copy(dst_hbm.at[pl.ds(wid * _CPW, _CPW)], dst_v)
    plsc.subcore_barrier()

    def body(j, carry):
        pltpu.sync_copy(ones_v, dacc_sh.at[dst_v.at[j]], add=True)
        return carry

    lax.fori_loop(0, _CPW, body, 0)
    plsc.subcore_barrier()
    pltpu.sync_copy(dacc_sh.at[pl.ds(sid * _RPT, _RPT)],
                    out_hbm.at[cid, pl.ds(sid * _RPT, _RPT)])


def _sc_deg(dst2d, zeros_rows, ones_rows):
    kern = functools.partial(
        pl.kernel,
        out_type=jax.ShapeDtypeStruct((_NC, _N_ACC, _D), jnp.float32),
        mesh=_mesh(),
        scratch_types=[
            pltpu.VMEM((_CPW, _K), jnp.int32),
            pltpu.VMEM((_K, _D), jnp.float32),
            pltpu.VMEM_SHARED((_N_ACC, _D), jnp.float32),
        ],
    )(_deg_body)
    return kern(dst2d, zeros_rows, ones_rows)


def _agg_body(h_hbm, src_hbm, dst_hbm, z_hbm, out_hbm,
              src_v, dst_v, rows_v, acc_sh, sem):
    cid = lax.axis_index("c")
    sid = lax.axis_index("s")
    pltpu.sync_copy(z_hbm, acc_sh.at[pl.ds(sid * _RPT, _RPT)])
    base = pl.multiple_of(
        jnp.where(cid == 0, sid * _C0PW, _C1_BASE + sid * _C1PW), 8)
    cnt = jnp.where(cid == 0, _C0PW, _C1PW)
    pltpu.sync_copy(src_hbm.at[pl.ds(base, _C0PW)], src_v)
    pltpu.sync_copy(dst_hbm.at[pl.ds(base, _C0PW)], dst_v)
    plsc.subcore_barrier()

    def body(j, carry):
        pltpu.async_copy(h_hbm.at[src_v.at[j]], rows_v, sem).wait()
        pltpu.sync_copy(rows_v, acc_sh.at[dst_v.at[j]], add=True)
        return carry

    lax.fori_loop(0, cnt, body, 0)
    plsc.subcore_barrier()
    pltpu.sync_copy(acc_sh.at[pl.ds(sid * _RPT, _RPT)],
                    out_hbm.at[cid, pl.ds(sid * _RPT, _RPT)])


def _sc_agg(h, src2d, dst2d, zeros_rows):
    kern = functools.partial(
        pl.kernel,
        out_type=jax.ShapeDtypeStruct((_NC, _N_ACC, _D), jnp.float32),
        mesh=_mesh(),
        scratch_types=[
            pltpu.VMEM((_C0PW, _K), jnp.int32),
            pltpu.VMEM((_C0PW, _K), jnp.int32),
            pltpu.VMEM((_K, _D), jnp.float32),
            pltpu.VMEM_SHARED((_N_ACC, _D), jnp.float32),
            pltpu.SemaphoreType.DMA,
        ],
    )(_agg_body)
    return kern(h, src2d, dst2d, zeros_rows)


def _mm_body(x_ref, w_ref, h_ref):
    h_ref[...] = jnp.dot(x_ref[...], w_ref[...],
                         preferred_element_type=jnp.float32,
                         precision=lax.Precision.HIGHEST)


def _tc_mm(x, w):
    return pl.pallas_call(
        _mm_body,
        out_shape=jax.ShapeDtypeStruct((_N, _D), jnp.float32),
    )(x, w)


def _scale_body(deg_ref, h_ref, dinv_ref, hp_ref):
    deg = deg_ref[0, :, 0:1] + deg_ref[1, :, 0:1] + 1.0
    dinv = lax.rsqrt(deg)
    dinv_ref[...] = dinv
    hp_ref[...] = h_ref[...] * dinv[:_N]


def _tc_scale(deg, h):
    return pl.pallas_call(
        _scale_body,
        out_shape=(jax.ShapeDtypeStruct((_N_ACC, 1), jnp.float32),
                   jax.ShapeDtypeStruct((_N, _D), jnp.float32)),
    )(deg, h)


def _mid_body(agg_ref, hp_ref, dinv_ref, b_ref, gw_ref, gb_ref, ga_ref,
              wn_ref, out_ref, hpn_ref):
    agg = agg_ref[0, :_N, :] + agg_ref[1, :_N, :] + hp_ref[...]
    t = agg * dinv_ref[:_N] + b_ref[...]
    mean = jnp.mean(t, axis=0, keepdims=True)
    u = t - ga_ref[...] * mean
    var = jnp.mean(u * u, axis=0, keepdims=True)
    g = u * lax.rsqrt(var + 1e-5) * gw_ref[...] + gb_ref[...]
    out_ref[...] = g
    e = jnp.where(g > 0, g, jnp.exp(jnp.minimum(g, 0.0)) - 1.0)
    hn = jnp.dot(e, wn_ref[...], preferred_element_type=jnp.float32,
                 precision=lax.Precision.HIGHEST)
    hpn_ref[...] = hn * dinv_ref[:_N]


def _tc_mid(agg, hp, dinv, b, gw, gb, ga, wnext):
    return pl.pallas_call(
        _mid_body,
        out_shape=(jax.ShapeDtypeStruct((_N, _D), jnp.float32),
                   jax.ShapeDtypeStruct((_N, _D), jnp.float32)),
    )(agg, hp, dinv, b, gw, gb, ga, wnext)


def _final_body(agg_ref, hp_ref, dinv_ref, b_ref, out_ref):
    agg = agg_ref[0, :_N, :] + agg_ref[1, :_N, :] + hp_ref[...]
    out_ref[...] = agg * dinv_ref[:_N] + b_ref[...]


def _tc_final(agg, hp, dinv, b):
    return pl.pallas_call(
        _final_body,
        out_shape=jax.ShapeDtypeStruct((_N, _D), jnp.float32),
    )(agg, hp, dinv, b)


def kernel(x, edge_index, W1, b1, W2, b2, W3, b3,
           gn_w1, gn_b1, gn_a1, gn_w2, gn_b2, gn_a2):
    src = edge_index[0]
    dst = edge_index[1]
    pad2 = _C_ALLOC * _K - _E
    src2d = jnp.concatenate(
        [src, jnp.zeros((pad2,), jnp.int32)]).reshape(_C_ALLOC, _K)
    dst2d = jnp.concatenate(
        [dst, jnp.full((pad2,), _N, jnp.int32)]).reshape(_C_ALLOC, _K)
    zeros_rows = jnp.zeros((_RPT, _D), jnp.float32)
    ones_rows = jnp.ones((_K, _D), jnp.float32)

    deg = _sc_deg(dst2d, zeros_rows, ones_rows)
    h1 = _tc_mm(x, W1)
    dinv, h1p = _tc_scale(deg, h1)

    agg1 = _sc_agg(h1p, src2d, dst2d, zeros_rows)
    out1, h2p = _tc_mid(agg1, h1p, dinv, b1.reshape(1, _D),
                        gn_w1.reshape(1, _D), gn_b1.reshape(1, _D),
                        gn_a1.reshape(1, _D), W2)

    agg2 = _sc_agg(h2p, src2d, dst2d, zeros_rows)
    out2, h3p = _tc_mid(agg2, h2p, dinv, b2.reshape(1, _D),
                        gn_w2.reshape(1, _D), gn_b2.reshape(1, _D),
                        gn_a2.reshape(1, _D), W3)

    agg3 = _sc_agg(h3p, src2d, dst2d, zeros_rows)
    out3 = _tc_final(agg3, h3p, dinv, b3.reshape(1, _D))

    return jnp.concatenate([out1, out2, out3], axis=-1)
